# initial kernel scaffold (unmeasured)
import jax
import jax.numpy as jnp
from jax import lax
from jax.experimental import pallas as pl
from jax.experimental.pallas import tpu as pltpu

T = 4096
D = 2048
F = 4096
E_LOC = 4
CAP = 1280
BT = 256
BF = 1024

_HBM = pltpu.MemorySpace.HBM


def _peer_id():
    mx = lax.axis_index("x")
    my = lax.axis_index("y")
    mz = lax.axis_index("z")
    return (mx, 1 - my, mz), my


def _exchange(x, assign):

    def body(x_ref, a_ref, xall_ref, aall_ref, loc_sems, send_sems, recv_sems):
        peer, my = _peer_id()
        barrier = pltpu.get_barrier_semaphore()
        pl.semaphore_signal(barrier, inc=1, device_id=peer,
                            device_id_type=pl.DeviceIdType.MESH)
        pl.semaphore_wait(barrier, 1)

        cp_x = pltpu.make_async_copy(x_ref, xall_ref.at[my], loc_sems.at[0])
        cp_a = pltpu.make_async_copy(a_ref, aall_ref.at[my], loc_sems.at[1])
        cp_x.start()
        cp_a.start()
        r_x = pltpu.make_async_remote_copy(
            src_ref=x_ref, dst_ref=xall_ref.at[my],
            send_sem=send_sems.at[0], recv_sem=recv_sems.at[0],
            device_id=peer, device_id_type=pl.DeviceIdType.MESH)
        r_a = pltpu.make_async_remote_copy(
            src_ref=a_ref, dst_ref=aall_ref.at[my],
            send_sem=send_sems.at[1], recv_sem=recv_sems.at[1],
            device_id=peer, device_id_type=pl.DeviceIdType.MESH)
        r_x.start()
        r_a.start()
        cp_x.wait()
        cp_a.wait()
        r_x.wait()
        r_a.wait()

    return pl.pallas_call(
        body,
        out_shape=[
            jax.ShapeDtypeStruct((2, T, D), jnp.float32),
            jax.ShapeDtypeStruct((2, T), jnp.int32),
        ],
        in_specs=[pl.BlockSpec(memory_space=_HBM),
                  pl.BlockSpec(memory_space=_HBM)],
        out_specs=[pl.BlockSpec(memory_space=_HBM),
                   pl.BlockSpec(memory_space=_HBM)],
        scratch_shapes=[
            pltpu.SemaphoreType.DMA((2,)),
            pltpu.SemaphoreType.DMA((2,)),
            pltpu.SemaphoreType.DMA((2,)),
        ],
        compiler_params=pltpu.CompilerParams(collective_id=0),
    )(x, assign)


def _grouped_ffn(x_grp, W1, W2):
    n_t = CAP // BT
    n_f = F // BF

    def body(x_ref, w1_ref, w2_ref, out_ref):
        f = pl.program_id(2)
        h = jnp.maximum(
            jnp.dot(x_ref[...], w1_ref[0], preferred_element_type=jnp.float32),
            0.0)
        acc = jnp.dot(h, w2_ref[0], preferred_element_type=jnp.float32)

        @pl.when(f == 0)
        def _():
            out_ref[...] = acc

        @pl.when(f != 0)
        def _():
            out_ref[...] = out_ref[...] + acc

    return pl.pallas_call(
        body,
        grid=(E_LOC, n_t, n_f),
        in_specs=[
            pl.BlockSpec((BT, D), lambda e, t, f: (e * (CAP // BT) + t, 0)),
            pl.BlockSpec((1, D, BF), lambda e, t, f: (e, 0, f)),
            pl.BlockSpec((1, BF, D), lambda e, t, f: (e, f, 0)),
        ],
        out_specs=pl.BlockSpec((BT, D), lambda e, t, f: (e * (CAP // BT) + t, 0)),
        out_shape=jax.ShapeDtypeStruct((E_LOC * CAP, D), jnp.float32),
    )(x_grp, W1, W2)


def _return_exchange(theirs):

    def body(src_ref, recv_ref, send_sem, recv_sem):
        peer, _ = _peer_id()
        barrier = pltpu.get_barrier_semaphore()
        pl.semaphore_signal(barrier, inc=1, device_id=peer,
                            device_id_type=pl.DeviceIdType.MESH)
        pl.semaphore_wait(barrier, 1)
        rdma = pltpu.make_async_remote_copy(
            src_ref=src_ref, dst_ref=recv_ref,
            send_sem=send_sem, recv_sem=recv_sem,
            device_id=peer, device_id_type=pl.DeviceIdType.MESH)
        rdma.start()
        rdma.wait()

    return pl.pallas_call(
        body,
        out_shape=jax.ShapeDtypeStruct((T, D), jnp.float32),
        in_specs=[pl.BlockSpec(memory_space=_HBM)],
        out_specs=pl.BlockSpec(memory_space=_HBM),
        scratch_shapes=[
            pltpu.SemaphoreType.DMA,
            pltpu.SemaphoreType.DMA,
        ],
        compiler_params=pltpu.CompilerParams(collective_id=1),
    )(theirs)


def kernel(x, assign, W1, W2):
    my_y = lax.axis_index("y")
    T2 = 2 * T

    xall, aall = _exchange(x, assign)
    x_all = xall.reshape(T2, D)
    a_all = aall.reshape(T2)

    le = a_all - my_y * E_LOC
    le = jnp.where((le >= 0) & (le < E_LOC), le, E_LOC).astype(jnp.int32)
    order = jnp.argsort(le, stable=True)
    s_le = le[order]
    sizes = jnp.bincount(le, length=E_LOC + 1)
    starts = (jnp.cumsum(sizes) - sizes).astype(jnp.int32)
    rank = jnp.arange(T2, dtype=jnp.int32) - starts[s_le]
    dest = jnp.where((s_le < E_LOC) & (rank < CAP),
                     s_le * CAP + rank, E_LOC * CAP)
    g = jnp.full((E_LOC * CAP + 1,), T2, dtype=jnp.int32)
    g = g.at[dest].set(order.astype(jnp.int32))[:E_LOC * CAP]

    x_pad = jnp.concatenate([x_all, jnp.zeros((1, D), jnp.float32)], axis=0)
    x_grp = x_pad[g]

    out_grp = _grouped_ffn(x_grp, W1, W2)

    out_all = jnp.zeros((T2 + 1, D), jnp.float32).at[g].set(out_grp)[:T2]
    mine = lax.dynamic_slice(out_all, (my_y * T, 0), (T, D))
    theirs = lax.dynamic_slice(out_all, ((1 - my_y) * T, 0), (T, D))

    recv = _return_exchange(theirs)
    return mine + recv


# baseline (device time: 3072008 ns/iter reference)
import jax
import jax.numpy as jnp
from jax import lax
from jax.experimental import pallas as pl
from jax.experimental.pallas import tpu as pltpu

T = 4096
D = 2048
F = 4096
E_LOC = 4
CAP = 1280
BT = 256
BF = 512

_HBM = pltpu.MemorySpace.HBM


def _peer_id():
    mx = lax.axis_index("x")
    my = lax.axis_index("y")
    mz = lax.axis_index("z")
    return (mx, 1 - my, mz), my


def _exchange(x, assign):

    def body(x_ref, a_ref, xall_ref, aall_ref, loc_sems, send_sems, recv_sems):
        peer, my = _peer_id()
        barrier = pltpu.get_barrier_semaphore()
        pl.semaphore_signal(barrier, inc=1, device_id=peer,
                            device_id_type=pl.DeviceIdType.MESH)
        pl.semaphore_wait(barrier, 1)

        cp_x = pltpu.make_async_copy(x_ref, xall_ref.at[my], loc_sems.at[0])
        cp_x.start()
        r_x = pltpu.make_async_remote_copy(
            src_ref=x_ref, dst_ref=xall_ref.at[my],
            send_sem=send_sems.at[0], recv_sem=recv_sems.at[0],
            device_id=peer, device_id_type=pl.DeviceIdType.MESH)
        r_a = pltpu.make_async_remote_copy(
            src_ref=a_ref, dst_ref=aall_ref.at[my],
            send_sem=send_sems.at[1], recv_sem=recv_sems.at[1],
            device_id=peer, device_id_type=pl.DeviceIdType.MESH)
        r_x.start()
        r_a.start()
        cp_x.wait()
        r_x.wait()
        r_a.wait()

    return pl.pallas_call(
        body,
        out_shape=[
            jax.ShapeDtypeStruct((2, T, D), jnp.float32),
            jax.ShapeDtypeStruct((2, 1, T), jnp.int32),
        ],
        in_specs=[pl.BlockSpec(memory_space=_HBM),
                  pl.BlockSpec(memory_space=_HBM)],
        out_specs=[pl.BlockSpec(memory_space=_HBM),
                   pl.BlockSpec(memory_space=_HBM)],
        scratch_shapes=[
            pltpu.SemaphoreType.DMA((2,)),
            pltpu.SemaphoreType.DMA((2,)),
            pltpu.SemaphoreType.DMA((2,)),
        ],
        compiler_params=pltpu.CompilerParams(collective_id=0),
    )(x, assign.reshape(1, T))


def _grouped_ffn(x_grp, W1, W2):
    n_t = CAP // BT
    n_f = F // BF

    def body(x_ref, w1_ref, w2_ref, out_ref):
        f = pl.program_id(2)
        h = jnp.maximum(
            jnp.dot(x_ref[...], w1_ref[0], preferred_element_type=jnp.float32),
            0.0)
        acc = jnp.dot(h, w2_ref[0], preferred_element_type=jnp.float32)

        @pl.when(f == 0)
        def _():
            out_ref[...] = acc

        @pl.when(f != 0)
        def _():
            out_ref[...] = out_ref[...] + acc

    return pl.pallas_call(
        body,
        grid=(E_LOC, n_t, n_f),
        in_specs=[
            pl.BlockSpec((BT, D), lambda e, t, f: (e * (CAP // BT) + t, 0)),
            pl.BlockSpec((1, D, BF), lambda e, t, f: (e, 0, f)),
            pl.BlockSpec((1, BF, D), lambda e, t, f: (e, f, 0)),
        ],
        out_specs=pl.BlockSpec((BT, D), lambda e, t, f: (e * (CAP // BT) + t, 0)),
        out_shape=jax.ShapeDtypeStruct((E_LOC * CAP, D), jnp.float32),
    )(x_grp, W1, W2)


def _return_exchange(theirs, collective_id=1):

    def body(src_ref, recv_ref, send_sem, recv_sem):
        peer, _ = _peer_id()
        barrier = pltpu.get_barrier_semaphore()
        pl.semaphore_signal(barrier, inc=1, device_id=peer,
                            device_id_type=pl.DeviceIdType.MESH)
        pl.semaphore_wait(barrier, 1)
        rdma = pltpu.make_async_remote_copy(
            src_ref=src_ref, dst_ref=recv_ref,
            send_sem=send_sem, recv_sem=recv_sem,
            device_id=peer, device_id_type=pl.DeviceIdType.MESH)
        rdma.start()
        rdma.wait()

    return pl.pallas_call(
        body,
        out_shape=jax.ShapeDtypeStruct((T, D), jnp.float32),
        in_specs=[pl.BlockSpec(memory_space=_HBM)],
        out_specs=pl.BlockSpec(memory_space=_HBM),
        scratch_shapes=[
            pltpu.SemaphoreType.DMA,
            pltpu.SemaphoreType.DMA,
        ],
        compiler_params=pltpu.CompilerParams(collective_id=collective_id),
    )(theirs)


def kernel(x, assign, W1, W2):
    my_y = lax.axis_index("y")
    T2 = 2 * T

    xall, aall = _exchange(x, assign)
    x_all = xall.reshape(T2, D)
    a_all = aall.at[my_y].set(assign.reshape(1, T)).reshape(T2)

    le = a_all - my_y * E_LOC
    le = jnp.where((le >= 0) & (le < E_LOC), le, E_LOC).astype(jnp.int32)
    order = jnp.argsort(le, stable=True)
    s_le = le[order]
    sizes = jnp.bincount(le, length=E_LOC + 1)
    starts = (jnp.cumsum(sizes) - sizes).astype(jnp.int32)
    rank = jnp.arange(T2, dtype=jnp.int32) - starts[s_le]
    dest = jnp.where((s_le < E_LOC) & (rank < CAP),
                     s_le * CAP + rank, E_LOC * CAP)
    g = jnp.full((E_LOC * CAP + 1,), T2, dtype=jnp.int32)
    g = g.at[dest].set(order.astype(jnp.int32))[:E_LOC * CAP]

    x_pad = jnp.concatenate([x_all, jnp.zeros((1, D), jnp.float32)], axis=0)
    x_grp = x_pad[g]

    out_grp = _grouped_ffn(x_grp, W1, W2)

    out_all = jnp.zeros((T2 + 1, D), jnp.float32).at[g].set(out_grp)[:T2]
    mine = lax.dynamic_slice(out_all, (my_y * T, 0), (T, D))
    theirs = lax.dynamic_slice(out_all, ((1 - my_y) * T, 0), (T, D))

    recv = _return_exchange(theirs)
    return mine + recv


# device time: 2260694 ns/iter; 1.3589x vs baseline; 1.3589x over previous
import jax
import jax.numpy as jnp
from jax import lax
from jax.experimental import pallas as pl
from jax.experimental.pallas import tpu as pltpu

T = 4096
D = 2048
F = 4096
E_LOC = 4
CAP = 1280
BT = 256
BF = 512

_HBM = pltpu.MemorySpace.HBM


def _peer_id():
    mx = lax.axis_index("x")
    my = lax.axis_index("y")
    mz = lax.axis_index("z")
    return (mx, 1 - my, mz), my


def _exchange(x, assign):

    def body(x_ref, a_ref, xall_ref, aall_ref, loc_sems, send_sems, recv_sems):
        peer, my = _peer_id()
        barrier = pltpu.get_barrier_semaphore()
        pl.semaphore_signal(barrier, inc=1, device_id=peer,
                            device_id_type=pl.DeviceIdType.MESH)
        pl.semaphore_wait(barrier, 1)

        cp_x = pltpu.make_async_copy(x_ref, xall_ref.at[my], loc_sems.at[0])
        cp_x.start()
        r_x = pltpu.make_async_remote_copy(
            src_ref=x_ref, dst_ref=xall_ref.at[my],
            send_sem=send_sems.at[0], recv_sem=recv_sems.at[0],
            device_id=peer, device_id_type=pl.DeviceIdType.MESH)
        r_a = pltpu.make_async_remote_copy(
            src_ref=a_ref, dst_ref=aall_ref.at[my],
            send_sem=send_sems.at[1], recv_sem=recv_sems.at[1],
            device_id=peer, device_id_type=pl.DeviceIdType.MESH)
        r_x.start()
        r_a.start()
        cp_x.wait()
        r_x.wait()
        r_a.wait()

    return pl.pallas_call(
        body,
        out_shape=[
            jax.ShapeDtypeStruct((2, T, D), jnp.float32),
            jax.ShapeDtypeStruct((2, 1, T), jnp.int32),
        ],
        in_specs=[pl.BlockSpec(memory_space=_HBM),
                  pl.BlockSpec(memory_space=_HBM)],
        out_specs=[pl.BlockSpec(memory_space=_HBM),
                   pl.BlockSpec(memory_space=_HBM)],
        scratch_shapes=[
            pltpu.SemaphoreType.DMA((2,)),
            pltpu.SemaphoreType.DMA((2,)),
            pltpu.SemaphoreType.DMA((2,)),
        ],
        compiler_params=pltpu.CompilerParams(collective_id=0),
    )(x, assign.reshape(1, T))


_N_GSEM = 8


def _grouped_ffn(g_idx, x_all, W1, W2):
    n_t = CAP // BT
    n_f = F // BF

    def body(g_ref, xall_ref, w1_ref, w2_ref, out_ref, xsc, sems):
        e = pl.program_id(0)
        t = pl.program_id(1)
        f = pl.program_id(2)
        off = (e * n_t + t) * BT

        @pl.when(f == 0)
        def _():
            for j in range(BT):
                pltpu.make_async_copy(
                    xall_ref.at[pl.ds(g_ref[off + j], 1), :],
                    xsc.at[pl.ds(j, 1), :],
                    sems.at[j % _N_GSEM],
                ).start()
            for j in range(BT):
                pltpu.make_async_copy(
                    xall_ref.at[pl.ds(0, 1), :],
                    xsc.at[pl.ds(0, 1), :],
                    sems.at[j % _N_GSEM],
                ).wait()

        h = jnp.maximum(
            jnp.dot(xsc[...], w1_ref[0], preferred_element_type=jnp.float32),
            0.0)
        acc = jnp.dot(h, w2_ref[0], preferred_element_type=jnp.float32)

        @pl.when(f == 0)
        def _():
            out_ref[...] = acc

        @pl.when(f != 0)
        def _():
            out_ref[...] = out_ref[...] + acc

    grid_spec = pltpu.PrefetchScalarGridSpec(
        num_scalar_prefetch=1,
        grid=(E_LOC, n_t, n_f),
        in_specs=[
            pl.BlockSpec(memory_space=pltpu.MemorySpace.HBM),
            pl.BlockSpec((1, D, BF), lambda e, t, f, g: (e, 0, f)),
            pl.BlockSpec((1, BF, D), lambda e, t, f, g: (e, f, 0)),
        ],
        out_specs=pl.BlockSpec((BT, D),
                               lambda e, t, f, g: (e * (CAP // BT) + t, 0)),
        scratch_shapes=[
            pltpu.VMEM((BT, D), jnp.float32),
            pltpu.SemaphoreType.DMA((_N_GSEM,)),
        ],
    )
    return pl.pallas_call(
        body,
        grid_spec=grid_spec,
        out_shape=jax.ShapeDtypeStruct((E_LOC * CAP, D), jnp.float32),
    )(g_idx, x_all, W1, W2)


def _return_exchange(theirs, collective_id=1):

    def body(src_ref, recv_ref, send_sem, recv_sem):
        peer, _ = _peer_id()
        barrier = pltpu.get_barrier_semaphore()
        pl.semaphore_signal(barrier, inc=1, device_id=peer,
                            device_id_type=pl.DeviceIdType.MESH)
        pl.semaphore_wait(barrier, 1)
        rdma = pltpu.make_async_remote_copy(
            src_ref=src_ref, dst_ref=recv_ref,
            send_sem=send_sem, recv_sem=recv_sem,
            device_id=peer, device_id_type=pl.DeviceIdType.MESH)
        rdma.start()
        rdma.wait()

    return pl.pallas_call(
        body,
        out_shape=jax.ShapeDtypeStruct((T, D), jnp.float32),
        in_specs=[pl.BlockSpec(memory_space=_HBM)],
        out_specs=pl.BlockSpec(memory_space=_HBM),
        scratch_shapes=[
            pltpu.SemaphoreType.DMA,
            pltpu.SemaphoreType.DMA,
        ],
        compiler_params=pltpu.CompilerParams(collective_id=collective_id),
    )(theirs)


def kernel(x, assign, W1, W2):
    my_y = lax.axis_index("y")
    T2 = 2 * T

    xall, aall = _exchange(x, assign)
    x_all = xall.reshape(T2, D)
    a_all = aall.at[my_y].set(assign.reshape(1, T)).reshape(T2)

    le = a_all - my_y * E_LOC
    le = jnp.where((le >= 0) & (le < E_LOC), le, E_LOC).astype(jnp.int32)
    order = jnp.argsort(le, stable=True)
    s_le = le[order]
    sizes = jnp.bincount(le, length=E_LOC + 1)
    starts = (jnp.cumsum(sizes) - sizes).astype(jnp.int32)
    rank = jnp.arange(T2, dtype=jnp.int32) - starts[s_le]
    dest = jnp.where((s_le < E_LOC) & (rank < CAP),
                     s_le * CAP + rank, E_LOC * CAP)
    g = jnp.full((E_LOC * CAP + 1,), T2, dtype=jnp.int32)
    g = g.at[dest].set(order.astype(jnp.int32))[:E_LOC * CAP]

    g_k = jnp.minimum(g, T2 - 1)
    out_grp = _grouped_ffn(g_k, x_all, W1, W2)

    out_all = jnp.zeros((T2 + 1, D), jnp.float32).at[g].set(out_grp)[:T2]
    mine = lax.dynamic_slice(out_all, (my_y * T, 0), (T, D))
    theirs = lax.dynamic_slice(out_all, ((1 - my_y) * T, 0), (T, D))

    recv = _return_exchange(theirs)
    return mine + recv
